# baseline (speedup 1.0000x reference)
import functools

import jax
import jax.numpy as jnp
from jax import lax
from jax.experimental import pallas as pl
from jax.experimental.pallas import tpu as pltpu
from jax.experimental.pallas import tpu_sc as plsc

NN = 10000
EE = 640000
DD = 128

NC = 2
NS = 16
CHUNK = 80
ROWS_STAGE = 50
N_OUTER = EE // (NS * ROWS_STAGE * CHUNK)
CHUNK_P = 64
PROP_STAGE = 25
PROP_EROWS = EE // CHUNK_P
PROP_OUTER = EE // (NS * PROP_STAGE * CHUNK_P)
NBUF = 5
LAG = 4
EROWS = EE // CHUNK
N_PER_TEC = NN // NS

_MESH = plsc.VectorSubcoreMesh(core_axis_name="c", subcore_axis_name="s")


@functools.partial(
    pl.kernel,
    out_type=jax.ShapeDtypeStruct((2, 2, 2 * NS, NN), jnp.float32),
    mesh=_MESH,
    scratch_types=[
        pltpu.VMEM((2, NN), jnp.float32),
        pltpu.VMEM((2, NN), jnp.float32),
        pltpu.VMEM((2, ROWS_STAGE, CHUNK), jnp.int32),
        pltpu.VMEM((2, ROWS_STAGE, CHUNK), jnp.int32),
        pltpu.SemaphoreType.DMA,
    ],
    compiler_params=pltpu.CompilerParams(
        use_tc_tiling_on_sc=False, needs_layout_passes=False),
)
def _sc_degrees(srcs, dsts, zeros1d, out, acco, acci, sidx, didx, stsem):
    c = lax.axis_index("c")
    s = lax.axis_index("s")
    pltpu.sync_copy(zeros1d, acco)
    pltpu.sync_copy(zeros1d, acci)
    ones = jnp.full((16,), 1.0, dtype=jnp.float32)
    pltpu.async_copy(srcs.at[c, pl.ds(s * (EROWS // NS), ROWS_STAGE)],
                     sidx.at[0], stsem)
    pltpu.async_copy(dsts.at[c, pl.ds(s * (EROWS // NS), ROWS_STAGE)],
                     didx.at[0], stsem)

    def body(o2, _):
        for par in (0, 1):
            o = 2 * o2 + par
            pltpu.make_async_copy(srcs.at[c, pl.ds(0, ROWS_STAGE)],
                                  sidx.at[par], stsem).wait()
            pltpu.make_async_copy(dsts.at[c, pl.ds(0, ROWS_STAGE)],
                                  didx.at[par], stsem).wait()

            @pl.when(o + 1 < N_OUTER)
            def _prefetch():
                nbase = s * (EROWS // NS) + (o + 1) * ROWS_STAGE
                pltpu.async_copy(srcs.at[c, pl.ds(nbase, ROWS_STAGE)],
                                 sidx.at[1 - par], stsem)
                pltpu.async_copy(dsts.at[c, pl.ds(nbase, ROWS_STAGE)],
                                 didx.at[1 - par], stsem)

            for j in range(ROWS_STAGE):
                k = j % 2
                for l in range(CHUNK // 16):
                    si = sidx[par, j, pl.ds(l * 16, 16)]
                    plsc.addupdate_scatter(acco.at[k], [si], ones)
                    di = didx[par, j, pl.ds(l * 16, 16)]
                    plsc.addupdate_scatter(acci.at[k], [di], ones)
        return _

    lax.fori_loop(0, N_OUTER // 2, body, None)
    pltpu.sync_copy(acco.at[0], out.at[c, 0, 2 * s])
    pltpu.sync_copy(acco.at[1], out.at[c, 0, 2 * s + 1])
    pltpu.sync_copy(acci.at[0], out.at[c, 1, 2 * s])
    pltpu.sync_copy(acci.at[1], out.at[c, 1, 2 * s + 1])


@functools.partial(
    pl.kernel,
    out_type=jax.ShapeDtypeStruct((2, NN, DD), jnp.float32),
    mesh=_MESH,
    scratch_types=[
        pltpu.VMEM_SHARED((NN, DD), jnp.float32),
        pltpu.VMEM((2, PROP_STAGE, CHUNK_P), jnp.int32),
        pltpu.VMEM((2, PROP_STAGE, CHUNK_P), jnp.int32),
        pltpu.VMEM((NBUF, CHUNK_P, DD), jnp.float32),
        pltpu.SemaphoreType.DMA,
        pltpu.SemaphoreType.DMA,
        pltpu.SemaphoreType.DMA,
        pltpu.SemaphoreType.DMA,
        pltpu.SemaphoreType.DMA,
        pltpu.SemaphoreType.DMA,
        pltpu.SemaphoreType.DMA,
        pltpu.SemaphoreType.DMA,
        pltpu.SemaphoreType.DMA,
        pltpu.SemaphoreType.DMA,
        pltpu.SemaphoreType.DMA,
    ],
    compiler_params=pltpu.CompilerParams(
        use_tc_tiling_on_sc=False, needs_layout_passes=False),
)
def _sc_prop(table, srcs, dsts, zeros2d, out, acc, sidx, didx, rows, stsem,
             g0, g1, g2, g3, g4, s0, s1, s2, s3, s4):
    c = lax.axis_index("c")
    s = lax.axis_index("s")
    gsem = [g0, g1, g2, g3, g4]
    ssem = [s0, s1, s2, s3, s4]
    pltpu.sync_copy(zeros2d, acc.at[pl.ds(s * N_PER_TEC, N_PER_TEC)])
    pltpu.async_copy(srcs.at[c, pl.ds(s * (PROP_EROWS // NS), PROP_STAGE)],
                     sidx.at[0], stsem)
    pltpu.async_copy(dsts.at[c, pl.ds(s * (PROP_EROWS // NS), PROP_STAGE)],
                     didx.at[0], stsem)
    plsc.subcore_barrier()

    def body(o, _):
        par = lax.rem(o, 2)
        pltpu.make_async_copy(srcs.at[c, pl.ds(0, PROP_STAGE)],
                              sidx.at[par], stsem).wait()
        pltpu.make_async_copy(dsts.at[c, pl.ds(0, PROP_STAGE)],
                              didx.at[par], stsem).wait()

        @pl.when(o + 1 < PROP_OUTER)
        def _prefetch():
            nbase = s * (PROP_EROWS // NS) + (o + 1) * PROP_STAGE
            pltpu.async_copy(srcs.at[c, pl.ds(nbase, PROP_STAGE)],
                             sidx.at[1 - par], stsem)
            pltpu.async_copy(dsts.at[c, pl.ds(nbase, PROP_STAGE)],
                             didx.at[1 - par], stsem)

        gd = [None] * NBUF
        sd = [None] * NBUF
        for j in range(PROP_STAGE + LAG):
            if j < PROP_STAGE:
                b = j % NBUF
                if j >= NBUF:
                    sd[b].wait()
                gd[b] = pltpu.async_copy(table.at[sidx.at[par, j]],
                                         rows.at[b], gsem[b])
            if j >= LAG:
                jj = j - LAG
                b2 = jj % NBUF
                gd[b2].wait()
                sd[b2] = pltpu.async_copy(rows.at[b2],
                                          acc.at[didx.at[par, jj]],
                                          ssem[b2], add=True)
        for b in range(NBUF):
            sd[(PROP_STAGE - NBUF + b) % NBUF].wait()
        return _

    lax.fori_loop(0, PROP_OUTER, body, None)
    plsc.subcore_barrier()
    pltpu.sync_copy(
        acc.at[pl.ds(s * N_PER_TEC, N_PER_TEC)],
        out.at[c, pl.ds(s * N_PER_TEC, N_PER_TEC)],
    )


def _tc1a_body(h_ref, w0a_ref, w0b_ref, hw_ref):
    h = h_ref[...]
    hw_ref[0:NN, :] = jnp.dot(h, w0a_ref[...],
                              preferred_element_type=jnp.float32)
    hw_ref[NN:2 * NN, :] = jnp.dot(h, w0b_ref[...],
                                   preferred_element_type=jnp.float32)


def _tc1a(h, w0a, w0b):
    return pl.pallas_call(
        _tc1a_body,
        out_shape=jax.ShapeDtypeStruct((2 * NN, DD), jnp.float32),
    )(h, w0a, w0b)


def _tc1b_body(hw_ref, degp_ref, hws_ref, so_ref, si_ref):
    deg = jnp.sum(degp_ref[...], axis=2)
    so = lax.rsqrt(jnp.maximum(deg[:, 0, :], 1.0))
    si = lax.rsqrt(jnp.maximum(deg[:, 1, :], 1.0))
    hws_ref[0:NN, :] = hw_ref[0:NN, :] * so[0][:, None]
    hws_ref[NN:2 * NN, :] = hw_ref[NN:2 * NN, :] * so[1][:, None]
    so_ref[...] = so
    si_ref[...] = si


def _tc1b(hw, degp):
    return pl.pallas_call(
        _tc1b_body,
        out_shape=(
            jax.ShapeDtypeStruct((2 * NN, DD), jnp.float32),
            jax.ShapeDtypeStruct((2, NN), jnp.float32),
            jax.ShapeDtypeStruct((2, NN), jnp.float32),
        ),
    )(hw, degp)


def _elu(x):
    return jnp.where(x > 0, x, jnp.exp(jnp.minimum(x, 0.0)) - 1.0)


def _tc2_body(agg_ref, si_ref, so_ref, b0a_ref, b0b_ref, w1a_ref, w1b_ref,
              hws_ref):
    si = si_ref[...]
    so = so_ref[...]
    x0 = _elu(agg_ref[0] * si[0][:, None] + b0a_ref[...][None, :])
    x1 = _elu(agg_ref[1] * si[1][:, None] + b0b_ref[...][None, :])
    hw0 = jnp.dot(x0, w1a_ref[...], preferred_element_type=jnp.float32)
    hw1 = jnp.dot(x1, w1b_ref[...], preferred_element_type=jnp.float32)
    hws_ref[0:NN, :] = hw0 * so[0][:, None]
    hws_ref[NN:2 * NN, :] = hw1 * so[1][:, None]


def _tc2(agg, si, so, b0a, b0b, w1a, w1b):
    return pl.pallas_call(
        _tc2_body,
        out_shape=jax.ShapeDtypeStruct((2 * NN, DD), jnp.float32),
    )(agg, si, so, b0a, b0b, w1a, w1b)


def _tc3_body(agg_ref, si_ref, b1a_ref, b1b_ref, wa1_ref, ba1_ref, wa2_ref,
              out_ref):
    si = si_ref[...]
    x0 = agg_ref[0] * si[0][:, None] + b1a_ref[...][None, :]
    x1 = agg_ref[1] * si[1][:, None] + b1b_ref[...][None, :]
    wa1 = wa1_ref[...]
    ba1 = ba1_ref[...][None, :]
    wa2 = wa2_ref[...][:, 0]
    t0 = jnp.tanh(jnp.dot(x0, wa1, preferred_element_type=jnp.float32) + ba1)
    t1 = jnp.tanh(jnp.dot(x1, wa1, preferred_element_type=jnp.float32) + ba1)
    m0 = jnp.mean(jnp.sum(t0 * wa2[None, :], axis=1))
    m1 = jnp.mean(jnp.sum(t1 * wa2[None, :], axis=1))
    mx = jnp.maximum(m0, m1)
    e0 = jnp.exp(m0 - mx)
    e1 = jnp.exp(m1 - mx)
    beta0 = e0 / (e0 + e1)
    beta1 = e1 / (e0 + e1)
    out_ref[...] = beta0 * x0 + beta1 * x1


def _tc3(agg, si, b1a, b1b, wa1, ba1, wa2):
    return pl.pallas_call(
        _tc3_body,
        out_shape=jax.ShapeDtypeStruct((NN, DD), jnp.float32),
    )(agg, si, b1a, b1b, wa1, ba1, wa2)


def kernel(h, edge_index_0, edge_index_1, W0_0, b0_0, W1_0, b1_0,
           W0_1, b0_1, W1_1, b1_1, Wa1, ba1, Wa2):
    src0, dst0 = edge_index_0[0], edge_index_0[1]
    src1, dst1 = edge_index_1[0], edge_index_1[1]
    srcs_plain = jnp.stack([src0, src1]).reshape(2, EROWS, CHUNK)
    srcs_adj = jnp.stack([src0, src1 + NN]).reshape(2, PROP_EROWS, CHUNK_P)
    dsts_stack = jnp.stack([dst0, dst1])
    dsts = dsts_stack.reshape(2, EROWS, CHUNK)
    dsts_p = dsts_stack.reshape(2, PROP_EROWS, CHUNK_P)
    zeros1d = jnp.zeros((2, NN), jnp.float32)
    zeros2d = jnp.zeros((N_PER_TEC, DD), jnp.float32)

    hw_un = _tc1a(h, W0_0, W0_1)
    degp = _sc_degrees(srcs_plain, dsts, zeros1d)
    hws0, so, si = _tc1b(hw_un, degp)
    agg0 = _sc_prop(hws0, srcs_adj, dsts_p, zeros2d)
    hws1 = _tc2(agg0, si, so, b0_0, b0_1, W1_0, W1_1)
    agg1 = _sc_prop(hws1, srcs_adj, dsts_p, zeros2d)
    return _tc3(agg1, si, b1_0, b1_1, Wa1, ba1, Wa2)


# speedup vs baseline: 1.0098x; 1.0098x over previous
import functools

import jax
import jax.numpy as jnp
from jax import lax
from jax.experimental import pallas as pl
from jax.experimental.pallas import tpu as pltpu
from jax.experimental.pallas import tpu_sc as plsc

NN = 10000
EE = 640000
DD = 128

NC = 2
NS = 16
CHUNK = 80
ROWS_STAGE = 50
N_OUTER = EE // (NS * ROWS_STAGE * CHUNK)
CHUNK_P = 80
PROP_STAGE = 25
PROP_EROWS = EE // CHUNK_P
PROP_OUTER = EE // (NS * PROP_STAGE * CHUNK_P)
NBUF = 4
LAG = 3
EROWS = EE // CHUNK
N_PER_TEC = NN // NS

_MESH = plsc.VectorSubcoreMesh(core_axis_name="c", subcore_axis_name="s")


@functools.partial(
    pl.kernel,
    out_type=jax.ShapeDtypeStruct((2, 2, 2 * NS, NN), jnp.float32),
    mesh=_MESH,
    scratch_types=[
        pltpu.VMEM((2, NN), jnp.float32),
        pltpu.VMEM((2, NN), jnp.float32),
        pltpu.VMEM((2, ROWS_STAGE, CHUNK), jnp.int32),
        pltpu.VMEM((2, ROWS_STAGE, CHUNK), jnp.int32),
        pltpu.SemaphoreType.DMA,
    ],
    compiler_params=pltpu.CompilerParams(
        use_tc_tiling_on_sc=False, needs_layout_passes=False),
)
def _sc_degrees(srcs, dsts, zeros1d, out, acco, acci, sidx, didx, stsem):
    c = lax.axis_index("c")
    s = lax.axis_index("s")
    pltpu.sync_copy(zeros1d, acco)
    pltpu.sync_copy(zeros1d, acci)
    ones = jnp.full((16,), 1.0, dtype=jnp.float32)
    pltpu.async_copy(srcs.at[c, pl.ds(s * (EROWS // NS), ROWS_STAGE)],
                     sidx.at[0], stsem)
    pltpu.async_copy(dsts.at[c, pl.ds(s * (EROWS // NS), ROWS_STAGE)],
                     didx.at[0], stsem)

    def body(o2, _):
        for par in (0, 1):
            o = 2 * o2 + par
            pltpu.make_async_copy(srcs.at[c, pl.ds(0, ROWS_STAGE)],
                                  sidx.at[par], stsem).wait()
            pltpu.make_async_copy(dsts.at[c, pl.ds(0, ROWS_STAGE)],
                                  didx.at[par], stsem).wait()

            @pl.when(o + 1 < N_OUTER)
            def _prefetch():
                nbase = s * (EROWS // NS) + (o + 1) * ROWS_STAGE
                pltpu.async_copy(srcs.at[c, pl.ds(nbase, ROWS_STAGE)],
                                 sidx.at[1 - par], stsem)
                pltpu.async_copy(dsts.at[c, pl.ds(nbase, ROWS_STAGE)],
                                 didx.at[1 - par], stsem)

            for j in range(ROWS_STAGE):
                k = j % 2
                for l in range(CHUNK // 16):
                    si = sidx[par, j, pl.ds(l * 16, 16)]
                    plsc.addupdate_scatter(acco.at[k], [si], ones)
                    di = didx[par, j, pl.ds(l * 16, 16)]
                    plsc.addupdate_scatter(acci.at[k], [di], ones)
        return _

    lax.fori_loop(0, N_OUTER // 2, body, None)
    pltpu.sync_copy(acco.at[0], out.at[c, 0, 2 * s])
    pltpu.sync_copy(acco.at[1], out.at[c, 0, 2 * s + 1])
    pltpu.sync_copy(acci.at[0], out.at[c, 1, 2 * s])
    pltpu.sync_copy(acci.at[1], out.at[c, 1, 2 * s + 1])


@functools.partial(
    pl.kernel,
    out_type=jax.ShapeDtypeStruct((2, NN, DD), jnp.float32),
    mesh=_MESH,
    scratch_types=[
        pltpu.VMEM_SHARED((NN, DD), jnp.float32),
        pltpu.VMEM((2, PROP_STAGE, CHUNK_P), jnp.int32),
        pltpu.VMEM((2, PROP_STAGE, CHUNK_P), jnp.int32),
        pltpu.VMEM((NBUF, CHUNK_P, DD), jnp.float32),
        pltpu.SemaphoreType.DMA,
        pltpu.SemaphoreType.DMA,
        pltpu.SemaphoreType.DMA,
        pltpu.SemaphoreType.DMA,
        pltpu.SemaphoreType.DMA,
        pltpu.SemaphoreType.DMA,
        pltpu.SemaphoreType.DMA,
        pltpu.SemaphoreType.DMA,
        pltpu.SemaphoreType.DMA,
    ],
    compiler_params=pltpu.CompilerParams(
        use_tc_tiling_on_sc=False, needs_layout_passes=False),
)
def _sc_prop(table, srcs, dsts, zeros2d, out, acc, sidx, didx, rows, stsem,
             g0, g1, g2, g3, s0, s1, s2, s3):
    c = lax.axis_index("c")
    s = lax.axis_index("s")
    gsem = [g0, g1, g2, g3]
    ssem = [s0, s1, s2, s3]
    pltpu.sync_copy(zeros2d, acc.at[pl.ds(s * N_PER_TEC, N_PER_TEC)])
    pltpu.async_copy(srcs.at[c, pl.ds(s * (PROP_EROWS // NS), PROP_STAGE)],
                     sidx.at[0], stsem)
    pltpu.async_copy(dsts.at[c, pl.ds(s * (PROP_EROWS // NS), PROP_STAGE)],
                     didx.at[0], stsem)
    plsc.subcore_barrier()

    def body(o, _):
        par = lax.rem(o, 2)
        pltpu.make_async_copy(srcs.at[c, pl.ds(0, PROP_STAGE)],
                              sidx.at[par], stsem).wait()
        pltpu.make_async_copy(dsts.at[c, pl.ds(0, PROP_STAGE)],
                              didx.at[par], stsem).wait()

        @pl.when(o + 1 < PROP_OUTER)
        def _prefetch():
            nbase = s * (PROP_EROWS // NS) + (o + 1) * PROP_STAGE
            pltpu.async_copy(srcs.at[c, pl.ds(nbase, PROP_STAGE)],
                             sidx.at[1 - par], stsem)
            pltpu.async_copy(dsts.at[c, pl.ds(nbase, PROP_STAGE)],
                             didx.at[1 - par], stsem)

        gd = [None] * NBUF
        sd = [None] * NBUF
        for j in range(PROP_STAGE + LAG):
            if j < PROP_STAGE:
                b = j % NBUF
                if j >= NBUF:
                    sd[b].wait()
                gd[b] = pltpu.async_copy(table.at[sidx.at[par, j]],
                                         rows.at[b], gsem[b])
            if j >= LAG:
                jj = j - LAG
                b2 = jj % NBUF
                gd[b2].wait()
                sd[b2] = pltpu.async_copy(rows.at[b2],
                                          acc.at[didx.at[par, jj]],
                                          ssem[b2], add=True)
        for b in range(NBUF):
            sd[(PROP_STAGE - NBUF + b) % NBUF].wait()
        return _

    lax.fori_loop(0, PROP_OUTER, body, None)
    plsc.subcore_barrier()
    pltpu.sync_copy(
        acc.at[pl.ds(s * N_PER_TEC, N_PER_TEC)],
        out.at[c, pl.ds(s * N_PER_TEC, N_PER_TEC)],
    )


def _tc1a_body(h_ref, w0a_ref, w0b_ref, hw_ref):
    h = h_ref[...]
    hw_ref[0:NN, :] = jnp.dot(h, w0a_ref[...],
                              preferred_element_type=jnp.float32)
    hw_ref[NN:2 * NN, :] = jnp.dot(h, w0b_ref[...],
                                   preferred_element_type=jnp.float32)


def _tc1a(h, w0a, w0b):
    return pl.pallas_call(
        _tc1a_body,
        out_shape=jax.ShapeDtypeStruct((2 * NN, DD), jnp.float32),
    )(h, w0a, w0b)


def _tc1b_body(hw_ref, degp_ref, hws_ref, so_ref, si_ref):
    deg = jnp.sum(degp_ref[...], axis=2)
    so = lax.rsqrt(jnp.maximum(deg[:, 0, :], 1.0))
    si = lax.rsqrt(jnp.maximum(deg[:, 1, :], 1.0))
    hws_ref[0:NN, :] = hw_ref[0:NN, :] * so[0][:, None]
    hws_ref[NN:2 * NN, :] = hw_ref[NN:2 * NN, :] * so[1][:, None]
    so_ref[...] = so
    si_ref[...] = si


def _tc1b(hw, degp):
    return pl.pallas_call(
        _tc1b_body,
        out_shape=(
            jax.ShapeDtypeStruct((2 * NN, DD), jnp.float32),
            jax.ShapeDtypeStruct((2, NN), jnp.float32),
            jax.ShapeDtypeStruct((2, NN), jnp.float32),
        ),
    )(hw, degp)


def _elu(x):
    return jnp.where(x > 0, x, jnp.exp(jnp.minimum(x, 0.0)) - 1.0)


def _tc2_body(agg_ref, si_ref, so_ref, b0a_ref, b0b_ref, w1a_ref, w1b_ref,
              hws_ref):
    si = si_ref[...]
    so = so_ref[...]
    x0 = _elu(agg_ref[0] * si[0][:, None] + b0a_ref[...][None, :])
    x1 = _elu(agg_ref[1] * si[1][:, None] + b0b_ref[...][None, :])
    hw0 = jnp.dot(x0, w1a_ref[...], preferred_element_type=jnp.float32)
    hw1 = jnp.dot(x1, w1b_ref[...], preferred_element_type=jnp.float32)
    hws_ref[0:NN, :] = hw0 * so[0][:, None]
    hws_ref[NN:2 * NN, :] = hw1 * so[1][:, None]


def _tc2(agg, si, so, b0a, b0b, w1a, w1b):
    return pl.pallas_call(
        _tc2_body,
        out_shape=jax.ShapeDtypeStruct((2 * NN, DD), jnp.float32),
    )(agg, si, so, b0a, b0b, w1a, w1b)


def _tc3_body(agg_ref, si_ref, b1a_ref, b1b_ref, wa1_ref, ba1_ref, wa2_ref,
              out_ref):
    si = si_ref[...]
    x0 = agg_ref[0] * si[0][:, None] + b1a_ref[...][None, :]
    x1 = agg_ref[1] * si[1][:, None] + b1b_ref[...][None, :]
    wa1 = wa1_ref[...]
    ba1 = ba1_ref[...][None, :]
    wa2 = wa2_ref[...][:, 0]
    t0 = jnp.tanh(jnp.dot(x0, wa1, preferred_element_type=jnp.float32) + ba1)
    t1 = jnp.tanh(jnp.dot(x1, wa1, preferred_element_type=jnp.float32) + ba1)
    m0 = jnp.mean(jnp.sum(t0 * wa2[None, :], axis=1))
    m1 = jnp.mean(jnp.sum(t1 * wa2[None, :], axis=1))
    mx = jnp.maximum(m0, m1)
    e0 = jnp.exp(m0 - mx)
    e1 = jnp.exp(m1 - mx)
    beta0 = e0 / (e0 + e1)
    beta1 = e1 / (e0 + e1)
    out_ref[...] = beta0 * x0 + beta1 * x1


def _tc3(agg, si, b1a, b1b, wa1, ba1, wa2):
    return pl.pallas_call(
        _tc3_body,
        out_shape=jax.ShapeDtypeStruct((NN, DD), jnp.float32),
    )(agg, si, b1a, b1b, wa1, ba1, wa2)


def kernel(h, edge_index_0, edge_index_1, W0_0, b0_0, W1_0, b1_0,
           W0_1, b0_1, W1_1, b1_1, Wa1, ba1, Wa2):
    src0, dst0 = edge_index_0[0], edge_index_0[1]
    src1, dst1 = edge_index_1[0], edge_index_1[1]
    srcs_plain = jnp.stack([src0, src1]).reshape(2, EROWS, CHUNK)
    srcs_adj = jnp.stack([src0, src1 + NN]).reshape(2, PROP_EROWS, CHUNK_P)
    dsts_stack = jnp.stack([dst0, dst1])
    dsts = dsts_stack.reshape(2, EROWS, CHUNK)
    dsts_p = dsts_stack.reshape(2, PROP_EROWS, CHUNK_P)
    zeros1d = jnp.zeros((2, NN), jnp.float32)
    zeros2d = jnp.zeros((N_PER_TEC, DD), jnp.float32)

    hw_un = _tc1a(h, W0_0, W0_1)
    degp = _sc_degrees(srcs_plain, dsts, zeros1d)
    hws0, so, si = _tc1b(hw_un, degp)
    agg0 = _sc_prop(hws0, srcs_adj, dsts_p, zeros2d)
    hws1 = _tc2(agg0, si, so, b0_0, b0_1, W1_0, W1_1)
    agg1 = _sc_prop(hws1, srcs_adj, dsts_p, zeros2d)
    return _tc3(agg1, si, b1_0, b1_1, Wa1, ba1, Wa2)


# speedup vs baseline: 1.0315x; 1.0215x over previous
import functools

import jax
import jax.numpy as jnp
from jax import lax
from jax.experimental import pallas as pl
from jax.experimental.pallas import tpu as pltpu
from jax.experimental.pallas import tpu_sc as plsc

NN = 10000
EE = 640000
DD = 128

NC = 2
NS = 16
CHUNK = 80
ROWS_STAGE = 50
N_OUTER = EE // (NS * ROWS_STAGE * CHUNK)
CHUNK_P = 80
PROP_STAGE = 25
PROP_EROWS = EE // CHUNK_P
PROP_OUTER = EE // (NS * PROP_STAGE * CHUNK_P)
NBUF = 4
LAG = 3
EROWS = EE // CHUNK
N_PER_TEC = NN // NS

_MESH = plsc.VectorSubcoreMesh(core_axis_name="c", subcore_axis_name="s")


@functools.partial(
    pl.kernel,
    out_type=jax.ShapeDtypeStruct((2, 2, 4 * NS, NN), jnp.float32),
    mesh=_MESH,
    scratch_types=[
        pltpu.VMEM((4, NN), jnp.float32),
        pltpu.VMEM((4, NN), jnp.float32),
        pltpu.VMEM((2, ROWS_STAGE, CHUNK), jnp.int32),
        pltpu.VMEM((2, ROWS_STAGE, CHUNK), jnp.int32),
        pltpu.SemaphoreType.DMA,
    ],
    compiler_params=pltpu.CompilerParams(
        use_tc_tiling_on_sc=False, needs_layout_passes=False),
)
def _sc_degrees(srcs, dsts, zeros1d, out, acco, acci, sidx, didx, stsem):
    c = lax.axis_index("c")
    s = lax.axis_index("s")
    pltpu.sync_copy(zeros1d, acco)
    pltpu.sync_copy(zeros1d, acci)
    ones = jnp.full((16,), 1.0, dtype=jnp.float32)
    pltpu.async_copy(srcs.at[c, pl.ds(s * (EROWS // NS), ROWS_STAGE)],
                     sidx.at[0], stsem)
    pltpu.async_copy(dsts.at[c, pl.ds(s * (EROWS // NS), ROWS_STAGE)],
                     didx.at[0], stsem)

    def body(o2, _):
        for par in (0, 1):
            o = 2 * o2 + par
            pltpu.make_async_copy(srcs.at[c, pl.ds(0, ROWS_STAGE)],
                                  sidx.at[par], stsem).wait()
            pltpu.make_async_copy(dsts.at[c, pl.ds(0, ROWS_STAGE)],
                                  didx.at[par], stsem).wait()

            @pl.when(o + 1 < N_OUTER)
            def _prefetch():
                nbase = s * (EROWS // NS) + (o + 1) * ROWS_STAGE
                pltpu.async_copy(srcs.at[c, pl.ds(nbase, ROWS_STAGE)],
                                 sidx.at[1 - par], stsem)
                pltpu.async_copy(dsts.at[c, pl.ds(nbase, ROWS_STAGE)],
                                 didx.at[1 - par], stsem)

            for j in range(ROWS_STAGE):
                for l in range(CHUNK // 16):
                    k = (j * (CHUNK // 16) + l) % 4
                    si = sidx[par, j, pl.ds(l * 16, 16)]
                    plsc.addupdate_scatter(acco.at[k], [si], ones)
                    di = didx[par, j, pl.ds(l * 16, 16)]
                    plsc.addupdate_scatter(acci.at[k], [di], ones)
        return _

    lax.fori_loop(0, N_OUTER // 2, body, None)
    for k in range(4):
        pltpu.sync_copy(acco.at[k], out.at[c, 0, 4 * s + k])
        pltpu.sync_copy(acci.at[k], out.at[c, 1, 4 * s + k])


@functools.partial(
    pl.kernel,
    out_type=jax.ShapeDtypeStruct((2, NN, DD), jnp.float32),
    mesh=_MESH,
    scratch_types=[
        pltpu.VMEM_SHARED((NN, DD), jnp.float32),
        pltpu.VMEM((2, PROP_STAGE, CHUNK_P), jnp.int32),
        pltpu.VMEM((2, PROP_STAGE, CHUNK_P), jnp.int32),
        pltpu.VMEM((NBUF, CHUNK_P, DD), jnp.float32),
        pltpu.SemaphoreType.DMA,
        pltpu.SemaphoreType.DMA,
        pltpu.SemaphoreType.DMA,
        pltpu.SemaphoreType.DMA,
        pltpu.SemaphoreType.DMA,
        pltpu.SemaphoreType.DMA,
        pltpu.SemaphoreType.DMA,
        pltpu.SemaphoreType.DMA,
        pltpu.SemaphoreType.DMA,
    ],
    compiler_params=pltpu.CompilerParams(
        use_tc_tiling_on_sc=False, needs_layout_passes=False),
)
def _sc_prop(table, srcs, dsts, zeros2d, out, acc, sidx, didx, rows, stsem,
             g0, g1, g2, g3, s0, s1, s2, s3):
    c = lax.axis_index("c")
    s = lax.axis_index("s")
    tbl = table.at[c]
    gsem = [g0, g1, g2, g3]
    ssem = [s0, s1, s2, s3]
    pltpu.async_copy(srcs.at[c, pl.ds(s * (PROP_EROWS // NS), PROP_STAGE)],
                     sidx.at[0], stsem)
    pltpu.async_copy(dsts.at[c, pl.ds(s * (PROP_EROWS // NS), PROP_STAGE)],
                     didx.at[0], stsem)
    pltpu.sync_copy(zeros2d, acc.at[pl.ds(s * N_PER_TEC, N_PER_TEC)])
    plsc.subcore_barrier()

    def body(o, _):
        par = lax.rem(o, 2)
        pltpu.make_async_copy(srcs.at[c, pl.ds(0, PROP_STAGE)],
                              sidx.at[par], stsem).wait()
        pltpu.make_async_copy(dsts.at[c, pl.ds(0, PROP_STAGE)],
                              didx.at[par], stsem).wait()

        @pl.when(o + 1 < PROP_OUTER)
        def _prefetch():
            nbase = s * (PROP_EROWS // NS) + (o + 1) * PROP_STAGE
            pltpu.async_copy(srcs.at[c, pl.ds(nbase, PROP_STAGE)],
                             sidx.at[1 - par], stsem)
            pltpu.async_copy(dsts.at[c, pl.ds(nbase, PROP_STAGE)],
                             didx.at[1 - par], stsem)

        gd = [None] * NBUF
        sd = [None] * NBUF
        for j in range(PROP_STAGE + LAG):
            if j < PROP_STAGE:
                b = j % NBUF
                if j >= NBUF:
                    sd[b].wait()
                gd[b] = pltpu.async_copy(tbl.at[sidx.at[par, j]],
                                         rows.at[b], gsem[b])
            if j >= LAG:
                jj = j - LAG
                b2 = jj % NBUF
                gd[b2].wait()
                sd[b2] = pltpu.async_copy(rows.at[b2],
                                          acc.at[didx.at[par, jj]],
                                          ssem[b2], add=True)
        for b in range(NBUF):
            sd[(PROP_STAGE - NBUF + b) % NBUF].wait()
        return _

    lax.fori_loop(0, PROP_OUTER, body, None)
    plsc.subcore_barrier()
    pltpu.sync_copy(
        acc.at[pl.ds(s * N_PER_TEC, N_PER_TEC)],
        out.at[c, pl.ds(s * N_PER_TEC, N_PER_TEC)],
    )


def _tc1a_body(h_ref, w0a_ref, w0b_ref, hw_ref):
    h = h_ref[...]
    hw_ref[0:NN, :] = jnp.dot(h, w0a_ref[...],
                              preferred_element_type=jnp.float32)
    hw_ref[NN:2 * NN, :] = jnp.dot(h, w0b_ref[...],
                                   preferred_element_type=jnp.float32)


def _tc1a(h, w0a, w0b):
    return pl.pallas_call(
        _tc1a_body,
        out_shape=jax.ShapeDtypeStruct((2 * NN, DD), jnp.float32),
    )(h, w0a, w0b)


def _tc1b_body(hw_ref, degp_ref, hws_ref, so_ref, si_ref):
    deg = jnp.sum(degp_ref[...], axis=2)
    so = lax.rsqrt(jnp.maximum(deg[:, 0, :], 1.0))
    si = lax.rsqrt(jnp.maximum(deg[:, 1, :], 1.0))
    hws_ref[0:NN, :] = hw_ref[0:NN, :] * so[0][:, None]
    hws_ref[NN:2 * NN, :] = hw_ref[NN:2 * NN, :] * so[1][:, None]
    so_ref[...] = so
    si_ref[...] = si


def _tc1b(hw, degp):
    return pl.pallas_call(
        _tc1b_body,
        out_shape=(
            jax.ShapeDtypeStruct((2 * NN, DD), jnp.float32),
            jax.ShapeDtypeStruct((2, NN), jnp.float32),
            jax.ShapeDtypeStruct((2, NN), jnp.float32),
        ),
    )(hw, degp)


def _elu(x):
    return jnp.where(x > 0, x, jnp.exp(jnp.minimum(x, 0.0)) - 1.0)


def _tc2_body(agg_ref, si_ref, so_ref, b0a_ref, b0b_ref, w1a_ref, w1b_ref,
              hws_ref):
    si = si_ref[...]
    so = so_ref[...]
    x0 = _elu(agg_ref[0] * si[0][:, None] + b0a_ref[...][None, :])
    x1 = _elu(agg_ref[1] * si[1][:, None] + b0b_ref[...][None, :])
    hw0 = jnp.dot(x0, w1a_ref[...], preferred_element_type=jnp.float32)
    hw1 = jnp.dot(x1, w1b_ref[...], preferred_element_type=jnp.float32)
    hws_ref[0:NN, :] = hw0 * so[0][:, None]
    hws_ref[NN:2 * NN, :] = hw1 * so[1][:, None]


def _tc2(agg, si, so, b0a, b0b, w1a, w1b):
    return pl.pallas_call(
        _tc2_body,
        out_shape=jax.ShapeDtypeStruct((2 * NN, DD), jnp.float32),
    )(agg, si, so, b0a, b0b, w1a, w1b)


def _tc3_body(agg_ref, si_ref, b1a_ref, b1b_ref, wa1_ref, ba1_ref, wa2_ref,
              out_ref):
    si = si_ref[...]
    x0 = agg_ref[0] * si[0][:, None] + b1a_ref[...][None, :]
    x1 = agg_ref[1] * si[1][:, None] + b1b_ref[...][None, :]
    wa1 = wa1_ref[...]
    ba1 = ba1_ref[...][None, :]
    wa2 = wa2_ref[...][:, 0]
    t0 = jnp.tanh(jnp.dot(x0, wa1, preferred_element_type=jnp.float32) + ba1)
    t1 = jnp.tanh(jnp.dot(x1, wa1, preferred_element_type=jnp.float32) + ba1)
    m0 = jnp.mean(jnp.sum(t0 * wa2[None, :], axis=1))
    m1 = jnp.mean(jnp.sum(t1 * wa2[None, :], axis=1))
    mx = jnp.maximum(m0, m1)
    e0 = jnp.exp(m0 - mx)
    e1 = jnp.exp(m1 - mx)
    beta0 = e0 / (e0 + e1)
    beta1 = e1 / (e0 + e1)
    out_ref[...] = beta0 * x0 + beta1 * x1


def _tc3(agg, si, b1a, b1b, wa1, ba1, wa2):
    return pl.pallas_call(
        _tc3_body,
        out_shape=jax.ShapeDtypeStruct((NN, DD), jnp.float32),
    )(agg, si, b1a, b1b, wa1, ba1, wa2)


def kernel(h, edge_index_0, edge_index_1, W0_0, b0_0, W1_0, b1_0,
           W0_1, b0_1, W1_1, b1_1, Wa1, ba1, Wa2):
    src0, dst0 = edge_index_0[0], edge_index_0[1]
    src1, dst1 = edge_index_1[0], edge_index_1[1]
    srcs_stack = jnp.stack([src0, src1])
    srcs_plain = srcs_stack.reshape(2, EROWS, CHUNK)
    srcs_p = srcs_stack.reshape(2, PROP_EROWS, CHUNK_P)
    dsts_stack = jnp.stack([dst0, dst1])
    dsts = dsts_stack.reshape(2, EROWS, CHUNK)
    dsts_p = dsts_stack.reshape(2, PROP_EROWS, CHUNK_P)
    zeros1d = jnp.zeros((4, NN), jnp.float32)
    zeros2d = jnp.zeros((N_PER_TEC, DD), jnp.float32)

    hw_un = _tc1a(h, W0_0, W0_1)
    degp = _sc_degrees(srcs_plain, dsts, zeros1d)
    hws0, so, si = _tc1b(hw_un, degp)
    agg0 = _sc_prop(hws0.reshape(2, NN, DD), srcs_p, dsts_p, zeros2d)
    hws1 = _tc2(agg0, si, so, b0_0, b0_1, W1_0, W1_1)
    agg1 = _sc_prop(hws1.reshape(2, NN, DD), srcs_p, dsts_p, zeros2d)
    return _tc3(agg1, si, b1_0, b1_1, Wa1, ba1, Wa2)


# speedup vs baseline: 1.0457x; 1.0138x over previous
import functools

import jax
import jax.numpy as jnp
from jax import lax
from jax.experimental import pallas as pl
from jax.experimental.pallas import tpu as pltpu
from jax.experimental.pallas import tpu_sc as plsc

NN = 10000
EE = 640000
DD = 128

NC = 2
NS = 16
CHUNK = 80
ROWS_STAGE = 50
N_OUTER = EE // (NS * ROWS_STAGE * CHUNK)
CHUNK_P = 80
PROP_STAGE = 25
PROP_EROWS = EE // CHUNK_P
PROP_OUTER = EE // (NS * PROP_STAGE * CHUNK_P)
NBUF = 4
LAG = 3
EROWS = EE // CHUNK
N_PER_TEC = NN // NS

_MESH = plsc.VectorSubcoreMesh(core_axis_name="c", subcore_axis_name="s")


@functools.partial(
    pl.kernel,
    out_type=jax.ShapeDtypeStruct((2, 2, 4 * NS, NN), jnp.float32),
    mesh=_MESH,
    scratch_types=[
        pltpu.VMEM((4, NN), jnp.float32),
        pltpu.VMEM((4, NN), jnp.float32),
        pltpu.VMEM((2, ROWS_STAGE, CHUNK), jnp.int32),
        pltpu.VMEM((2, ROWS_STAGE, CHUNK), jnp.int32),
        pltpu.SemaphoreType.DMA,
    ],
    compiler_params=pltpu.CompilerParams(
        use_tc_tiling_on_sc=False, needs_layout_passes=False),
)
def _sc_degrees(s0r, s1r, d0r, d1r, zeros1d, out, acco, acci, sidx, didx, stsem):
    c = lax.axis_index("c")
    s = lax.axis_index("s")
    pltpu.sync_copy(zeros1d, acco)
    pltpu.sync_copy(zeros1d, acci)
    ones = jnp.full((16,), 1.0, dtype=jnp.float32)
    @pl.when(c == 0)
    def _pro0():
        pltpu.async_copy(s0r.at[pl.ds(s * (EROWS // NS), ROWS_STAGE)],
                         sidx.at[0], stsem)
        pltpu.async_copy(d0r.at[pl.ds(s * (EROWS // NS), ROWS_STAGE)],
                         didx.at[0], stsem)

    @pl.when(c == 1)
    def _pro1():
        pltpu.async_copy(s1r.at[pl.ds(s * (EROWS // NS), ROWS_STAGE)],
                         sidx.at[0], stsem)
        pltpu.async_copy(d1r.at[pl.ds(s * (EROWS // NS), ROWS_STAGE)],
                         didx.at[0], stsem)

    def body(o2, _):
        for par in (0, 1):
            o = 2 * o2 + par
            pltpu.make_async_copy(s0r.at[pl.ds(0, ROWS_STAGE)],
                                  sidx.at[par], stsem).wait()
            pltpu.make_async_copy(d0r.at[pl.ds(0, ROWS_STAGE)],
                                  didx.at[par], stsem).wait()

            @pl.when((o + 1 < N_OUTER) & (c == 0))
            def _prefetch0():
                nbase = s * (EROWS // NS) + (o + 1) * ROWS_STAGE
                pltpu.async_copy(s0r.at[pl.ds(nbase, ROWS_STAGE)],
                                 sidx.at[1 - par], stsem)
                pltpu.async_copy(d0r.at[pl.ds(nbase, ROWS_STAGE)],
                                 didx.at[1 - par], stsem)

            @pl.when((o + 1 < N_OUTER) & (c == 1))
            def _prefetch1():
                nbase = s * (EROWS // NS) + (o + 1) * ROWS_STAGE
                pltpu.async_copy(s1r.at[pl.ds(nbase, ROWS_STAGE)],
                                 sidx.at[1 - par], stsem)
                pltpu.async_copy(d1r.at[pl.ds(nbase, ROWS_STAGE)],
                                 didx.at[1 - par], stsem)

            for j in range(ROWS_STAGE):
                for l in range(CHUNK // 16):
                    k = (j * (CHUNK // 16) + l) % 4
                    si = sidx[par, j, pl.ds(l * 16, 16)]
                    plsc.addupdate_scatter(acco.at[k], [si], ones)
                    di = didx[par, j, pl.ds(l * 16, 16)]
                    plsc.addupdate_scatter(acci.at[k], [di], ones)
        return _

    lax.fori_loop(0, N_OUTER // 2, body, None)
    for k in range(4):
        pltpu.sync_copy(acco.at[k], out.at[c, 0, 4 * s + k])
        pltpu.sync_copy(acci.at[k], out.at[c, 1, 4 * s + k])


@functools.partial(
    pl.kernel,
    out_type=jax.ShapeDtypeStruct((2, NN, DD), jnp.float32),
    mesh=_MESH,
    scratch_types=[
        pltpu.VMEM_SHARED((NN, DD), jnp.float32),
        pltpu.VMEM((2, PROP_STAGE, CHUNK_P), jnp.int32),
        pltpu.VMEM((2, PROP_STAGE, CHUNK_P), jnp.int32),
        pltpu.VMEM((NBUF, CHUNK_P, DD), jnp.float32),
        pltpu.SemaphoreType.DMA,
        pltpu.SemaphoreType.DMA,
        pltpu.SemaphoreType.DMA,
        pltpu.SemaphoreType.DMA,
        pltpu.SemaphoreType.DMA,
        pltpu.SemaphoreType.DMA,
        pltpu.SemaphoreType.DMA,
        pltpu.SemaphoreType.DMA,
        pltpu.SemaphoreType.DMA,
    ],
    compiler_params=pltpu.CompilerParams(
        use_tc_tiling_on_sc=False, needs_layout_passes=False),
)
def _sc_prop(table, s0r, s1r, d0r, d1r, zeros2d, out, acc, sidx, didx, rows, stsem,
             g0, g1, g2, g3, s0, s1, s2, s3):
    c = lax.axis_index("c")
    s = lax.axis_index("s")
    tbl = table.at[c]
    gsem = [g0, g1, g2, g3]
    ssem = [s0, s1, s2, s3]
    @pl.when(c == 0)
    def _pro0():
        pltpu.async_copy(s0r.at[pl.ds(s * (PROP_EROWS // NS), PROP_STAGE)],
                         sidx.at[0], stsem)
        pltpu.async_copy(d0r.at[pl.ds(s * (PROP_EROWS // NS), PROP_STAGE)],
                         didx.at[0], stsem)

    @pl.when(c == 1)
    def _pro1():
        pltpu.async_copy(s1r.at[pl.ds(s * (PROP_EROWS // NS), PROP_STAGE)],
                         sidx.at[0], stsem)
        pltpu.async_copy(d1r.at[pl.ds(s * (PROP_EROWS // NS), PROP_STAGE)],
                         didx.at[0], stsem)
    pltpu.sync_copy(zeros2d, acc.at[pl.ds(s * N_PER_TEC, N_PER_TEC)])
    plsc.subcore_barrier()

    def body(o, _):
        par = lax.rem(o, 2)
        pltpu.make_async_copy(s0r.at[pl.ds(0, PROP_STAGE)],
                              sidx.at[par], stsem).wait()
        pltpu.make_async_copy(d0r.at[pl.ds(0, PROP_STAGE)],
                              didx.at[par], stsem).wait()

        @pl.when((o + 1 < PROP_OUTER) & (c == 0))
        def _prefetch0():
            nbase = s * (PROP_EROWS // NS) + (o + 1) * PROP_STAGE
            pltpu.async_copy(s0r.at[pl.ds(nbase, PROP_STAGE)],
                             sidx.at[1 - par], stsem)
            pltpu.async_copy(d0r.at[pl.ds(nbase, PROP_STAGE)],
                             didx.at[1 - par], stsem)

        @pl.when((o + 1 < PROP_OUTER) & (c == 1))
        def _prefetch1():
            nbase = s * (PROP_EROWS // NS) + (o + 1) * PROP_STAGE
            pltpu.async_copy(s1r.at[pl.ds(nbase, PROP_STAGE)],
                             sidx.at[1 - par], stsem)
            pltpu.async_copy(d1r.at[pl.ds(nbase, PROP_STAGE)],
                             didx.at[1 - par], stsem)

        gd = [None] * NBUF
        sd = [None] * NBUF
        for j in range(PROP_STAGE + LAG):
            if j < PROP_STAGE:
                b = j % NBUF
                if j >= NBUF:
                    sd[b].wait()
                gd[b] = pltpu.async_copy(tbl.at[sidx.at[par, j]],
                                         rows.at[b], gsem[b])
            if j >= LAG:
                jj = j - LAG
                b2 = jj % NBUF
                gd[b2].wait()
                sd[b2] = pltpu.async_copy(rows.at[b2],
                                          acc.at[didx.at[par, jj]],
                                          ssem[b2], add=True)
        for b in range(NBUF):
            sd[(PROP_STAGE - NBUF + b) % NBUF].wait()
        return _

    lax.fori_loop(0, PROP_OUTER, body, None)
    plsc.subcore_barrier()
    pltpu.sync_copy(
        acc.at[pl.ds(s * N_PER_TEC, N_PER_TEC)],
        out.at[c, pl.ds(s * N_PER_TEC, N_PER_TEC)],
    )


def _tc1a_body(h_ref, w0a_ref, w0b_ref, hw_ref):
    h = h_ref[...]
    hw_ref[0:NN, :] = jnp.dot(h, w0a_ref[...],
                              preferred_element_type=jnp.float32)
    hw_ref[NN:2 * NN, :] = jnp.dot(h, w0b_ref[...],
                                   preferred_element_type=jnp.float32)


def _tc1a(h, w0a, w0b):
    return pl.pallas_call(
        _tc1a_body,
        out_shape=jax.ShapeDtypeStruct((2 * NN, DD), jnp.float32),
    )(h, w0a, w0b)


def _tc1b_body(hw_ref, degp_ref, hws_ref, so_ref, si_ref):
    deg = jnp.sum(degp_ref[...], axis=2)
    so = lax.rsqrt(jnp.maximum(deg[:, 0, :], 1.0))
    si = lax.rsqrt(jnp.maximum(deg[:, 1, :], 1.0))
    hws_ref[0:NN, :] = hw_ref[0:NN, :] * so[0][:, None]
    hws_ref[NN:2 * NN, :] = hw_ref[NN:2 * NN, :] * so[1][:, None]
    so_ref[...] = so
    si_ref[...] = si


def _tc1b(hw, degp):
    return pl.pallas_call(
        _tc1b_body,
        out_shape=(
            jax.ShapeDtypeStruct((2 * NN, DD), jnp.float32),
            jax.ShapeDtypeStruct((2, NN), jnp.float32),
            jax.ShapeDtypeStruct((2, NN), jnp.float32),
        ),
    )(hw, degp)


def _elu(x):
    return jnp.where(x > 0, x, jnp.exp(jnp.minimum(x, 0.0)) - 1.0)


def _tc2_body(agg_ref, si_ref, so_ref, b0a_ref, b0b_ref, w1a_ref, w1b_ref,
              hws_ref):
    si = si_ref[...]
    so = so_ref[...]
    x0 = _elu(agg_ref[0] * si[0][:, None] + b0a_ref[...][None, :])
    x1 = _elu(agg_ref[1] * si[1][:, None] + b0b_ref[...][None, :])
    hw0 = jnp.dot(x0, w1a_ref[...], preferred_element_type=jnp.float32)
    hw1 = jnp.dot(x1, w1b_ref[...], preferred_element_type=jnp.float32)
    hws_ref[0:NN, :] = hw0 * so[0][:, None]
    hws_ref[NN:2 * NN, :] = hw1 * so[1][:, None]


def _tc2(agg, si, so, b0a, b0b, w1a, w1b):
    return pl.pallas_call(
        _tc2_body,
        out_shape=jax.ShapeDtypeStruct((2 * NN, DD), jnp.float32),
    )(agg, si, so, b0a, b0b, w1a, w1b)


def _tc3_body(agg_ref, si_ref, b1a_ref, b1b_ref, wa1_ref, ba1_ref, wa2_ref,
              out_ref):
    si = si_ref[...]
    x0 = agg_ref[0] * si[0][:, None] + b1a_ref[...][None, :]
    x1 = agg_ref[1] * si[1][:, None] + b1b_ref[...][None, :]
    wa1 = wa1_ref[...]
    ba1 = ba1_ref[...][None, :]
    wa2 = wa2_ref[...][:, 0]
    t0 = jnp.tanh(jnp.dot(x0, wa1, preferred_element_type=jnp.float32) + ba1)
    t1 = jnp.tanh(jnp.dot(x1, wa1, preferred_element_type=jnp.float32) + ba1)
    m0 = jnp.mean(jnp.sum(t0 * wa2[None, :], axis=1))
    m1 = jnp.mean(jnp.sum(t1 * wa2[None, :], axis=1))
    mx = jnp.maximum(m0, m1)
    e0 = jnp.exp(m0 - mx)
    e1 = jnp.exp(m1 - mx)
    beta0 = e0 / (e0 + e1)
    beta1 = e1 / (e0 + e1)
    out_ref[...] = beta0 * x0 + beta1 * x1


def _tc3(agg, si, b1a, b1b, wa1, ba1, wa2):
    return pl.pallas_call(
        _tc3_body,
        out_shape=jax.ShapeDtypeStruct((NN, DD), jnp.float32),
    )(agg, si, b1a, b1b, wa1, ba1, wa2)


def kernel(h, edge_index_0, edge_index_1, W0_0, b0_0, W1_0, b1_0,
           W0_1, b0_1, W1_1, b1_1, Wa1, ba1, Wa2):
    src0, dst0 = edge_index_0[0], edge_index_0[1]
    src1, dst1 = edge_index_1[0], edge_index_1[1]
    s0r = src0.reshape(EROWS, CHUNK)
    s1r = src1.reshape(EROWS, CHUNK)
    d0r = dst0.reshape(EROWS, CHUNK)
    d1r = dst1.reshape(EROWS, CHUNK)
    zeros1d = jnp.zeros((4, NN), jnp.float32)
    zeros2d = jnp.zeros((N_PER_TEC, DD), jnp.float32)

    hw_un = _tc1a(h, W0_0, W0_1)
    degp = _sc_degrees(s0r, s1r, d0r, d1r, zeros1d)
    hws0, so, si = _tc1b(hw_un, degp)
    agg0 = _sc_prop(hws0.reshape(2, NN, DD), s0r, s1r, d0r, d1r, zeros2d)
    hws1 = _tc2(agg0, si, so, b0_0, b0_1, W1_0, W1_1)
    agg1 = _sc_prop(hws1.reshape(2, NN, DD), s0r, s1r, d0r, d1r, zeros2d)
    return _tc3(agg1, si, b1_0, b1_1, Wa1, ba1, Wa2)


# speedup vs baseline: 1.0520x; 1.0060x over previous
import functools

import jax
import jax.numpy as jnp
from jax import lax
from jax.experimental import pallas as pl
from jax.experimental.pallas import tpu as pltpu
from jax.experimental.pallas import tpu_sc as plsc

NN = 10000
EE = 640000
DD = 128

NC = 2
NS = 16
CHUNK = 80
ROWS_STAGE = 50
N_OUTER = EE // (NS * ROWS_STAGE * CHUNK)
CHUNK_P = 80
PROP_STAGE = 25
PROP_EROWS = EE // CHUNK_P
PROP_OUTER = EE // (NS * PROP_STAGE * CHUNK_P)
NBUF = 4
LAG = 3
EROWS = EE // CHUNK
N_PER_TEC = NN // NS

_MESH = plsc.VectorSubcoreMesh(core_axis_name="c", subcore_axis_name="s")


@functools.partial(
    pl.kernel,
    out_type=jax.ShapeDtypeStruct((2, 2, 4 * NS, NN), jnp.float32),
    mesh=_MESH,
    scratch_types=[
        pltpu.VMEM((4, NN), jnp.float32),
        pltpu.VMEM((4, NN), jnp.float32),
        pltpu.VMEM((2, ROWS_STAGE, CHUNK), jnp.int32),
        pltpu.VMEM((2, ROWS_STAGE, CHUNK), jnp.int32),
        pltpu.SemaphoreType.DMA,
    ],
    compiler_params=pltpu.CompilerParams(
        use_tc_tiling_on_sc=False, needs_layout_passes=False),
)
def _sc_degrees(s0r, s1r, d0r, d1r, zeros1d, out, acco, acci, sidx, didx, stsem):
    c = lax.axis_index("c")
    s = lax.axis_index("s")
    pltpu.sync_copy(zeros1d, acco)
    pltpu.sync_copy(zeros1d, acci)
    ones = jnp.full((16,), 1.0, dtype=jnp.float32)
    @pl.when(c == 0)
    def _pro0():
        pltpu.async_copy(s0r.at[pl.ds(s * (EROWS // NS), ROWS_STAGE)],
                         sidx.at[0], stsem)
        pltpu.async_copy(d0r.at[pl.ds(s * (EROWS // NS), ROWS_STAGE)],
                         didx.at[0], stsem)

    @pl.when(c == 1)
    def _pro1():
        pltpu.async_copy(s1r.at[pl.ds(s * (EROWS // NS), ROWS_STAGE)],
                         sidx.at[0], stsem)
        pltpu.async_copy(d1r.at[pl.ds(s * (EROWS // NS), ROWS_STAGE)],
                         didx.at[0], stsem)

    def body(o2, _):
        for par in (0, 1):
            o = 2 * o2 + par
            pltpu.make_async_copy(s0r.at[pl.ds(0, ROWS_STAGE)],
                                  sidx.at[par], stsem).wait()
            pltpu.make_async_copy(d0r.at[pl.ds(0, ROWS_STAGE)],
                                  didx.at[par], stsem).wait()

            @pl.when((o + 1 < N_OUTER) & (c == 0))
            def _prefetch0():
                nbase = s * (EROWS // NS) + (o + 1) * ROWS_STAGE
                pltpu.async_copy(s0r.at[pl.ds(nbase, ROWS_STAGE)],
                                 sidx.at[1 - par], stsem)
                pltpu.async_copy(d0r.at[pl.ds(nbase, ROWS_STAGE)],
                                 didx.at[1 - par], stsem)

            @pl.when((o + 1 < N_OUTER) & (c == 1))
            def _prefetch1():
                nbase = s * (EROWS // NS) + (o + 1) * ROWS_STAGE
                pltpu.async_copy(s1r.at[pl.ds(nbase, ROWS_STAGE)],
                                 sidx.at[1 - par], stsem)
                pltpu.async_copy(d1r.at[pl.ds(nbase, ROWS_STAGE)],
                                 didx.at[1 - par], stsem)

            for j in range(ROWS_STAGE):
                for l in range(CHUNK // 16):
                    k = (j * (CHUNK // 16) + l) % 4
                    si = sidx[par, j, pl.ds(l * 16, 16)]
                    plsc.addupdate_scatter(acco.at[k], [si], ones)
                    di = didx[par, j, pl.ds(l * 16, 16)]
                    plsc.addupdate_scatter(acci.at[k], [di], ones)
        return _

    lax.fori_loop(0, N_OUTER // 2, body, None)
    for k in range(4):
        pltpu.sync_copy(acco.at[k], out.at[c, 0, 4 * s + k])
        pltpu.sync_copy(acci.at[k], out.at[c, 1, 4 * s + k])


@functools.partial(
    pl.kernel,
    out_type=jax.ShapeDtypeStruct((2, NN, DD), jnp.float32),
    mesh=_MESH,
    scratch_types=[
        pltpu.VMEM_SHARED((NN, DD), jnp.float32),
        pltpu.VMEM((2, PROP_STAGE, CHUNK_P), jnp.int32),
        pltpu.VMEM((2, PROP_STAGE, CHUNK_P), jnp.int32),
        pltpu.VMEM((NBUF, CHUNK_P, DD), jnp.float32),
        pltpu.SemaphoreType.DMA,
        pltpu.SemaphoreType.DMA,
        pltpu.SemaphoreType.DMA,
        pltpu.SemaphoreType.DMA,
        pltpu.SemaphoreType.DMA,
        pltpu.SemaphoreType.DMA,
        pltpu.SemaphoreType.DMA,
        pltpu.SemaphoreType.DMA,
        pltpu.SemaphoreType.DMA,
    ],
    compiler_params=pltpu.CompilerParams(
        use_tc_tiling_on_sc=False, needs_layout_passes=False),
)
def _sc_prop(table, s0r, s1r, d0r, d1r, zeros2d, out, acc, sidx, didx, rows, stsem,
             g0, g1, g2, g3, s0, s1, s2, s3):
    c = lax.axis_index("c")
    s = lax.axis_index("s")
    tbl = table.at[c]
    gsem = [g0, g1, g2, g3]
    ssem = [s0, s1, s2, s3]
    @pl.when(c == 0)
    def _pro0():
        pltpu.async_copy(s0r.at[pl.ds(s * (PROP_EROWS // NS), PROP_STAGE)],
                         sidx.at[0], stsem)
        pltpu.async_copy(d0r.at[pl.ds(s * (PROP_EROWS // NS), PROP_STAGE)],
                         didx.at[0], stsem)

    @pl.when(c == 1)
    def _pro1():
        pltpu.async_copy(s1r.at[pl.ds(s * (PROP_EROWS // NS), PROP_STAGE)],
                         sidx.at[0], stsem)
        pltpu.async_copy(d1r.at[pl.ds(s * (PROP_EROWS // NS), PROP_STAGE)],
                         didx.at[0], stsem)
    pltpu.sync_copy(zeros2d, acc.at[pl.ds(s * N_PER_TEC, N_PER_TEC)])
    plsc.subcore_barrier()

    def body(o, _):
        par = lax.rem(o, 2)
        pltpu.make_async_copy(s0r.at[pl.ds(0, PROP_STAGE)],
                              sidx.at[par], stsem).wait()
        pltpu.make_async_copy(d0r.at[pl.ds(0, PROP_STAGE)],
                              didx.at[par], stsem).wait()

        @pl.when((o + 1 < PROP_OUTER) & (c == 0))
        def _prefetch0():
            nbase = s * (PROP_EROWS // NS) + (o + 1) * PROP_STAGE
            pltpu.async_copy(s0r.at[pl.ds(nbase, PROP_STAGE)],
                             sidx.at[1 - par], stsem)
            pltpu.async_copy(d0r.at[pl.ds(nbase, PROP_STAGE)],
                             didx.at[1 - par], stsem)

        @pl.when((o + 1 < PROP_OUTER) & (c == 1))
        def _prefetch1():
            nbase = s * (PROP_EROWS // NS) + (o + 1) * PROP_STAGE
            pltpu.async_copy(s1r.at[pl.ds(nbase, PROP_STAGE)],
                             sidx.at[1 - par], stsem)
            pltpu.async_copy(d1r.at[pl.ds(nbase, PROP_STAGE)],
                             didx.at[1 - par], stsem)

        gd = [None] * NBUF
        sd = [None] * NBUF
        for j in range(PROP_STAGE + LAG):
            if j < PROP_STAGE:
                b = j % NBUF
                if j >= NBUF:
                    sd[b].wait()
                gd[b] = pltpu.async_copy(tbl.at[sidx.at[par, j]],
                                         rows.at[b], gsem[b])
            if j >= LAG:
                jj = j - LAG
                b2 = jj % NBUF
                gd[b2].wait()
                sd[b2] = pltpu.async_copy(rows.at[b2],
                                          acc.at[didx.at[par, jj]],
                                          ssem[b2], add=True)
        for b in range(NBUF):
            sd[(PROP_STAGE - NBUF + b) % NBUF].wait()
        return _

    lax.fori_loop(0, PROP_OUTER, body, None)
    plsc.subcore_barrier()
    pltpu.sync_copy(
        acc.at[pl.ds(s * N_PER_TEC, N_PER_TEC)],
        out.at[c, pl.ds(s * N_PER_TEC, N_PER_TEC)],
    )


def _tc1_body(h_ref, w0a_ref, w0b_ref, degp_ref, hws_ref, so_ref, si_ref):
    deg = jnp.sum(degp_ref[...], axis=2)
    so = lax.rsqrt(jnp.maximum(deg[:, 0, :], 1.0))
    si = lax.rsqrt(jnp.maximum(deg[:, 1, :], 1.0))
    h = h_ref[...]
    hw0 = jnp.dot(h, w0a_ref[...], preferred_element_type=jnp.float32)
    hw1 = jnp.dot(h, w0b_ref[...], preferred_element_type=jnp.float32)
    hws_ref[0:NN, :] = hw0 * so[0][:, None]
    hws_ref[NN:2 * NN, :] = hw1 * so[1][:, None]
    so_ref[...] = so
    si_ref[...] = si


def _tc1(h, w0a, w0b, degp):
    return pl.pallas_call(
        _tc1_body,
        out_shape=(
            jax.ShapeDtypeStruct((2 * NN, DD), jnp.float32),
            jax.ShapeDtypeStruct((2, NN), jnp.float32),
            jax.ShapeDtypeStruct((2, NN), jnp.float32),
        ),
    )(h, w0a, w0b, degp)


def _elu(x):
    return jnp.where(x > 0, x, jnp.exp(jnp.minimum(x, 0.0)) - 1.0)


def _tc2_body(agg_ref, si_ref, so_ref, b0a_ref, b0b_ref, w1a_ref, w1b_ref,
              hws_ref):
    si = si_ref[...]
    so = so_ref[...]
    x0 = _elu(agg_ref[0] * si[0][:, None] + b0a_ref[...][None, :])
    x1 = _elu(agg_ref[1] * si[1][:, None] + b0b_ref[...][None, :])
    hw0 = jnp.dot(x0, w1a_ref[...], preferred_element_type=jnp.float32)
    hw1 = jnp.dot(x1, w1b_ref[...], preferred_element_type=jnp.float32)
    hws_ref[0:NN, :] = hw0 * so[0][:, None]
    hws_ref[NN:2 * NN, :] = hw1 * so[1][:, None]


def _tc2(agg, si, so, b0a, b0b, w1a, w1b):
    return pl.pallas_call(
        _tc2_body,
        out_shape=jax.ShapeDtypeStruct((2 * NN, DD), jnp.float32),
    )(agg, si, so, b0a, b0b, w1a, w1b)


def _tc3_body(agg_ref, si_ref, b1a_ref, b1b_ref, wa1_ref, ba1_ref, wa2_ref,
              out_ref):
    si = si_ref[...]
    x0 = agg_ref[0] * si[0][:, None] + b1a_ref[...][None, :]
    x1 = agg_ref[1] * si[1][:, None] + b1b_ref[...][None, :]
    wa1 = wa1_ref[...]
    ba1 = ba1_ref[...][None, :]
    wa2 = wa2_ref[...][:, 0]
    t0 = jnp.tanh(jnp.dot(x0, wa1, preferred_element_type=jnp.float32) + ba1)
    t1 = jnp.tanh(jnp.dot(x1, wa1, preferred_element_type=jnp.float32) + ba1)
    m0 = jnp.mean(jnp.sum(t0 * wa2[None, :], axis=1))
    m1 = jnp.mean(jnp.sum(t1 * wa2[None, :], axis=1))
    mx = jnp.maximum(m0, m1)
    e0 = jnp.exp(m0 - mx)
    e1 = jnp.exp(m1 - mx)
    beta0 = e0 / (e0 + e1)
    beta1 = e1 / (e0 + e1)
    out_ref[...] = beta0 * x0 + beta1 * x1


def _tc3(agg, si, b1a, b1b, wa1, ba1, wa2):
    return pl.pallas_call(
        _tc3_body,
        out_shape=jax.ShapeDtypeStruct((NN, DD), jnp.float32),
    )(agg, si, b1a, b1b, wa1, ba1, wa2)


def kernel(h, edge_index_0, edge_index_1, W0_0, b0_0, W1_0, b1_0,
           W0_1, b0_1, W1_1, b1_1, Wa1, ba1, Wa2):
    src0, dst0 = edge_index_0[0], edge_index_0[1]
    src1, dst1 = edge_index_1[0], edge_index_1[1]
    s0r = src0.reshape(EROWS, CHUNK)
    s1r = src1.reshape(EROWS, CHUNK)
    d0r = dst0.reshape(EROWS, CHUNK)
    d1r = dst1.reshape(EROWS, CHUNK)
    zeros1d = jnp.zeros((4, NN), jnp.float32)
    zeros2d = jnp.zeros((N_PER_TEC, DD), jnp.float32)

    degp = _sc_degrees(s0r, s1r, d0r, d1r, zeros1d)
    hws0, so, si = _tc1(h, W0_0, W0_1, degp)
    agg0 = _sc_prop(hws0.reshape(2, NN, DD), s0r, s1r, d0r, d1r, zeros2d)
    hws1 = _tc2(agg0, si, so, b0_0, b0_1, W1_0, W1_1)
    agg1 = _sc_prop(hws1.reshape(2, NN, DD), s0r, s1r, d0r, d1r, zeros2d)
    return _tc3(agg1, si, b1_0, b1_1, Wa1, ba1, Wa2)


# speedup vs baseline: 1.0590x; 1.0066x over previous
import functools

import jax
import jax.numpy as jnp
from jax import lax
from jax.experimental import pallas as pl
from jax.experimental.pallas import tpu as pltpu
from jax.experimental.pallas import tpu_sc as plsc

NN = 10000
EE = 640000
DD = 128

NC = 2
NS = 16
CHUNK = 80
ROWS_STAGE = 50
N_OUTER = EE // (NS * ROWS_STAGE * CHUNK)
CHUNK_P = 80
PROP_STAGE = 25
PROP_EROWS = EE // CHUNK_P
PROP_OUTER = EE // (NS * PROP_STAGE * CHUNK_P)
NBUF = 4
LAG = 3
EROWS = EE // CHUNK
N_PER_TEC = NN // NS

_MESH = plsc.VectorSubcoreMesh(core_axis_name="c", subcore_axis_name="s")


@functools.partial(
    pl.kernel,
    out_type=jax.ShapeDtypeStruct((2, 2, 4 * NS, NN), jnp.float32),
    mesh=_MESH,
    scratch_types=[
        pltpu.VMEM((4, NN), jnp.float32),
        pltpu.VMEM((4, NN), jnp.float32),
        pltpu.VMEM((2, ROWS_STAGE, CHUNK), jnp.int32),
        pltpu.VMEM((2, ROWS_STAGE, CHUNK), jnp.int32),
        pltpu.SemaphoreType.DMA,
    ],
    compiler_params=pltpu.CompilerParams(
        use_tc_tiling_on_sc=False, needs_layout_passes=False),
)
def _sc_degrees(s0r, s1r, d0r, d1r, zeros1d, out, acco, acci, sidx, didx, stsem):
    c = lax.axis_index("c")
    s = lax.axis_index("s")
    pltpu.sync_copy(zeros1d, acco)
    pltpu.sync_copy(zeros1d, acci)
    ones = jnp.full((16,), 1.0, dtype=jnp.float32)
    @pl.when(c == 0)
    def _pro0():
        pltpu.async_copy(s0r.at[pl.ds(s * (EROWS // NS), ROWS_STAGE)],
                         sidx.at[0], stsem)
        pltpu.async_copy(d0r.at[pl.ds(s * (EROWS // NS), ROWS_STAGE)],
                         didx.at[0], stsem)

    @pl.when(c == 1)
    def _pro1():
        pltpu.async_copy(s1r.at[pl.ds(s * (EROWS // NS), ROWS_STAGE)],
                         sidx.at[0], stsem)
        pltpu.async_copy(d1r.at[pl.ds(s * (EROWS // NS), ROWS_STAGE)],
                         didx.at[0], stsem)

    def body(o2, _):
        for par in (0, 1):
            o = 2 * o2 + par
            pltpu.make_async_copy(s0r.at[pl.ds(0, ROWS_STAGE)],
                                  sidx.at[par], stsem).wait()
            pltpu.make_async_copy(d0r.at[pl.ds(0, ROWS_STAGE)],
                                  didx.at[par], stsem).wait()

            @pl.when((o + 1 < N_OUTER) & (c == 0))
            def _prefetch0():
                nbase = s * (EROWS // NS) + (o + 1) * ROWS_STAGE
                pltpu.async_copy(s0r.at[pl.ds(nbase, ROWS_STAGE)],
                                 sidx.at[1 - par], stsem)
                pltpu.async_copy(d0r.at[pl.ds(nbase, ROWS_STAGE)],
                                 didx.at[1 - par], stsem)

            @pl.when((o + 1 < N_OUTER) & (c == 1))
            def _prefetch1():
                nbase = s * (EROWS // NS) + (o + 1) * ROWS_STAGE
                pltpu.async_copy(s1r.at[pl.ds(nbase, ROWS_STAGE)],
                                 sidx.at[1 - par], stsem)
                pltpu.async_copy(d1r.at[pl.ds(nbase, ROWS_STAGE)],
                                 didx.at[1 - par], stsem)

            for j in range(ROWS_STAGE):
                for l in range(CHUNK // 16):
                    k = (j * (CHUNK // 16) + l) % 4
                    si = sidx[par, j, pl.ds(l * 16, 16)]
                    plsc.addupdate_scatter(acco.at[k], [si], ones)
                    di = didx[par, j, pl.ds(l * 16, 16)]
                    plsc.addupdate_scatter(acci.at[k], [di], ones)
        return _

    lax.fori_loop(0, N_OUTER // 2, body, None)
    for k in range(4):
        pltpu.sync_copy(acco.at[k], out.at[c, 0, 4 * s + k])
        pltpu.sync_copy(acci.at[k], out.at[c, 1, 4 * s + k])


@functools.partial(
    pl.kernel,
    out_type=jax.ShapeDtypeStruct((2, NN, DD), jnp.float32),
    mesh=_MESH,
    scratch_types=[
        pltpu.VMEM_SHARED((NN, DD), jnp.float32),
        pltpu.VMEM((2, PROP_STAGE, CHUNK_P), jnp.int32),
        pltpu.VMEM((2, PROP_STAGE, CHUNK_P), jnp.int32),
        pltpu.VMEM((NBUF, CHUNK_P, DD), jnp.float32),
        pltpu.SemaphoreType.DMA,
        pltpu.SemaphoreType.DMA,
        pltpu.SemaphoreType.DMA,
        pltpu.SemaphoreType.DMA,
        pltpu.SemaphoreType.DMA,
        pltpu.SemaphoreType.DMA,
        pltpu.SemaphoreType.DMA,
        pltpu.SemaphoreType.DMA,
        pltpu.SemaphoreType.DMA,
    ],
    compiler_params=pltpu.CompilerParams(
        use_tc_tiling_on_sc=False, needs_layout_passes=False),
)
def _sc_prop(table, s0r, s1r, d0r, d1r, out, acc, sidx, didx, rows, stsem,
             g0, g1, g2, g3, s0, s1, s2, s3):
    c = lax.axis_index("c")
    s = lax.axis_index("s")
    tbl = table.at[c]
    gsem = [g0, g1, g2, g3]
    ssem = [s0, s1, s2, s3]
    @pl.when(c == 0)
    def _pro0():
        pltpu.async_copy(s0r.at[pl.ds(s * (PROP_EROWS // NS), PROP_STAGE)],
                         sidx.at[0], stsem)
        pltpu.async_copy(d0r.at[pl.ds(s * (PROP_EROWS // NS), PROP_STAGE)],
                         didx.at[0], stsem)

    @pl.when(c == 1)
    def _pro1():
        pltpu.async_copy(s1r.at[pl.ds(s * (PROP_EROWS // NS), PROP_STAGE)],
                         sidx.at[0], stsem)
        pltpu.async_copy(d1r.at[pl.ds(s * (PROP_EROWS // NS), PROP_STAGE)],
                         didx.at[0], stsem)
    zv = jnp.zeros((16,), dtype=jnp.float32)
    for i in range(CHUNK_P):
        for l in range(DD // 16):
            rows[0, i, pl.ds(l * 16, 16)] = zv
    for k in range(N_PER_TEC // CHUNK_P):
        pltpu.sync_copy(rows.at[0],
                        acc.at[pl.ds(s * N_PER_TEC + k * CHUNK_P, CHUNK_P)])
    rem = N_PER_TEC % CHUNK_P
    pltpu.sync_copy(rows.at[0, pl.ds(0, rem)],
                    acc.at[pl.ds(s * N_PER_TEC + N_PER_TEC - rem, rem)])
    plsc.subcore_barrier()

    def body(o, _):
        par = lax.rem(o, 2)
        pltpu.make_async_copy(s0r.at[pl.ds(0, PROP_STAGE)],
                              sidx.at[par], stsem).wait()
        pltpu.make_async_copy(d0r.at[pl.ds(0, PROP_STAGE)],
                              didx.at[par], stsem).wait()

        @pl.when((o + 1 < PROP_OUTER) & (c == 0))
        def _prefetch0():
            nbase = s * (PROP_EROWS // NS) + (o + 1) * PROP_STAGE
            pltpu.async_copy(s0r.at[pl.ds(nbase, PROP_STAGE)],
                             sidx.at[1 - par], stsem)
            pltpu.async_copy(d0r.at[pl.ds(nbase, PROP_STAGE)],
                             didx.at[1 - par], stsem)

        @pl.when((o + 1 < PROP_OUTER) & (c == 1))
        def _prefetch1():
            nbase = s * (PROP_EROWS // NS) + (o + 1) * PROP_STAGE
            pltpu.async_copy(s1r.at[pl.ds(nbase, PROP_STAGE)],
                             sidx.at[1 - par], stsem)
            pltpu.async_copy(d1r.at[pl.ds(nbase, PROP_STAGE)],
                             didx.at[1 - par], stsem)

        gd = [None] * NBUF
        sd = [None] * NBUF
        for j in range(PROP_STAGE + LAG):
            if j < PROP_STAGE:
                b = j % NBUF
                if j >= NBUF:
                    sd[b].wait()
                gd[b] = pltpu.async_copy(tbl.at[sidx.at[par, j]],
                                         rows.at[b], gsem[b])
            if j >= LAG:
                jj = j - LAG
                b2 = jj % NBUF
                gd[b2].wait()
                sd[b2] = pltpu.async_copy(rows.at[b2],
                                          acc.at[didx.at[par, jj]],
                                          ssem[b2], add=True)
        for b in range(NBUF):
            sd[(PROP_STAGE - NBUF + b) % NBUF].wait()
        return _

    lax.fori_loop(0, PROP_OUTER, body, None)
    plsc.subcore_barrier()
    pltpu.sync_copy(
        acc.at[pl.ds(s * N_PER_TEC, N_PER_TEC)],
        out.at[c, pl.ds(s * N_PER_TEC, N_PER_TEC)],
    )


def _tc1_body(h_ref, w0a_ref, w0b_ref, degp_ref, hws_ref, so_ref, si_ref):
    deg = jnp.sum(degp_ref[...], axis=2)
    so = lax.rsqrt(jnp.maximum(deg[:, 0, :], 1.0))
    si = lax.rsqrt(jnp.maximum(deg[:, 1, :], 1.0))
    h = h_ref[...]
    hw0 = jnp.dot(h, w0a_ref[...], preferred_element_type=jnp.float32)
    hw1 = jnp.dot(h, w0b_ref[...], preferred_element_type=jnp.float32)
    hws_ref[0:NN, :] = hw0 * so[0][:, None]
    hws_ref[NN:2 * NN, :] = hw1 * so[1][:, None]
    so_ref[...] = so
    si_ref[...] = si


def _tc1(h, w0a, w0b, degp):
    return pl.pallas_call(
        _tc1_body,
        out_shape=(
            jax.ShapeDtypeStruct((2 * NN, DD), jnp.float32),
            jax.ShapeDtypeStruct((2, NN), jnp.float32),
            jax.ShapeDtypeStruct((2, NN), jnp.float32),
        ),
    )(h, w0a, w0b, degp)


def _elu(x):
    return jnp.where(x > 0, x, jnp.exp(jnp.minimum(x, 0.0)) - 1.0)


def _tc2_body(agg_ref, si_ref, so_ref, b0a_ref, b0b_ref, w1a_ref, w1b_ref,
              hws_ref):
    si = si_ref[...]
    so = so_ref[...]
    x0 = _elu(agg_ref[0] * si[0][:, None] + b0a_ref[...][None, :])
    x1 = _elu(agg_ref[1] * si[1][:, None] + b0b_ref[...][None, :])
    hw0 = jnp.dot(x0, w1a_ref[...], preferred_element_type=jnp.float32)
    hw1 = jnp.dot(x1, w1b_ref[...], preferred_element_type=jnp.float32)
    hws_ref[0:NN, :] = hw0 * so[0][:, None]
    hws_ref[NN:2 * NN, :] = hw1 * so[1][:, None]


def _tc2(agg, si, so, b0a, b0b, w1a, w1b):
    return pl.pallas_call(
        _tc2_body,
        out_shape=jax.ShapeDtypeStruct((2 * NN, DD), jnp.float32),
    )(agg, si, so, b0a, b0b, w1a, w1b)


def _tc3_body(agg_ref, si_ref, b1a_ref, b1b_ref, wa1_ref, ba1_ref, wa2_ref,
              out_ref):
    si = si_ref[...]
    x0 = agg_ref[0] * si[0][:, None] + b1a_ref[...][None, :]
    x1 = agg_ref[1] * si[1][:, None] + b1b_ref[...][None, :]
    wa1 = wa1_ref[...]
    ba1 = ba1_ref[...][None, :]
    wa2 = wa2_ref[...][:, 0]
    t0 = jnp.tanh(jnp.dot(x0, wa1, preferred_element_type=jnp.float32) + ba1)
    t1 = jnp.tanh(jnp.dot(x1, wa1, preferred_element_type=jnp.float32) + ba1)
    m0 = jnp.mean(jnp.sum(t0 * wa2[None, :], axis=1))
    m1 = jnp.mean(jnp.sum(t1 * wa2[None, :], axis=1))
    mx = jnp.maximum(m0, m1)
    e0 = jnp.exp(m0 - mx)
    e1 = jnp.exp(m1 - mx)
    beta0 = e0 / (e0 + e1)
    beta1 = e1 / (e0 + e1)
    out_ref[...] = beta0 * x0 + beta1 * x1


def _tc3(agg, si, b1a, b1b, wa1, ba1, wa2):
    return pl.pallas_call(
        _tc3_body,
        out_shape=jax.ShapeDtypeStruct((NN, DD), jnp.float32),
    )(agg, si, b1a, b1b, wa1, ba1, wa2)


def kernel(h, edge_index_0, edge_index_1, W0_0, b0_0, W1_0, b1_0,
           W0_1, b0_1, W1_1, b1_1, Wa1, ba1, Wa2):
    src0, dst0 = edge_index_0[0], edge_index_0[1]
    src1, dst1 = edge_index_1[0], edge_index_1[1]
    s0r = src0.reshape(EROWS, CHUNK)
    s1r = src1.reshape(EROWS, CHUNK)
    d0r = dst0.reshape(EROWS, CHUNK)
    d1r = dst1.reshape(EROWS, CHUNK)
    zeros1d = jnp.zeros((4, NN), jnp.float32)

    degp = _sc_degrees(s0r, s1r, d0r, d1r, zeros1d)
    hws0, so, si = _tc1(h, W0_0, W0_1, degp)
    agg0 = _sc_prop(hws0.reshape(2, NN, DD), s0r, s1r, d0r, d1r)
    hws1 = _tc2(agg0, si, so, b0_0, b0_1, W1_0, W1_1)
    agg1 = _sc_prop(hws1.reshape(2, NN, DD), s0r, s1r, d0r, d1r)
    return _tc3(agg1, si, b1_0, b1_1, Wa1, ba1, Wa2)
